# baseline (speedup 1.0000x reference)
import functools

import jax
import jax.numpy as jnp
from jax import lax
from jax.experimental import pallas as pl
from jax.experimental.pallas import tpu as pltpu
from jax.experimental.pallas import tpu_sc as plsc

_NC = 2
_NS = 16
_LANES = 16


def kernel(x, pe):
    batch, seq_len, dim = x.shape
    rows = batch * seq_len
    nw = _NC * _NS
    rows_per_w = rows // nw
    ch = 16
    n_ch = rows_per_w // ch
    chunk_el = ch * dim
    unroll = 8
    groups = chunk_el // (_LANES * unroll)

    mesh = plsc.VectorSubcoreMesh(core_axis_name="c", subcore_axis_name="s")

    @functools.partial(
        pl.kernel,
        mesh=mesh,
        out_type=jax.ShapeDtypeStruct((rows * dim,), jnp.float32),
        scratch_types=[
            pltpu.VMEM((chunk_el,), jnp.float32),
            pltpu.VMEM((chunk_el,), jnp.float32),
            pltpu.SemaphoreType.DMA,
            pltpu.SemaphoreType.DMA,
            pltpu.SemaphoreType.DMA,
        ],
    )
    def sc_add(x_hbm, pe_hbm, out_hbm, xb, peb, semx, semp, semo):
        wid = lax.axis_index("s") * _NC + lax.axis_index("c")
        e0 = wid * rows_per_w * dim
        s0 = (wid * rows_per_w) % seq_len

        def chunk_body(i, carry):
            xoff = e0 + i * chunk_el
            poff = (s0 + i * ch) * dim
            cx = pltpu.async_copy(x_hbm.at[pl.ds(xoff, chunk_el)], xb, semx)
            cp = pltpu.async_copy(pe_hbm.at[pl.ds(poff, chunk_el)], peb, semp)
            cx.wait()
            cp.wait()

            def vec_body(j, c2):
                base = j * (_LANES * unroll)
                for u in range(unroll):
                    o = base + u * _LANES
                    xb[pl.ds(o, _LANES)] = xb[pl.ds(o, _LANES)] + peb[pl.ds(o, _LANES)]
                return c2

            lax.fori_loop(0, groups, vec_body, 0)
            co = pltpu.async_copy(xb, out_hbm.at[pl.ds(xoff, chunk_el)], semo)
            co.wait()
            return carry

        lax.fori_loop(0, n_ch, chunk_body, 0)

    out = sc_add(x.reshape(rows * dim), pe.reshape(seq_len * dim))
    return out.reshape(batch, seq_len, dim)


# speedup vs baseline: 1.3163x; 1.3163x over previous
import functools

import jax
import jax.numpy as jnp
from jax import lax
from jax.experimental import pallas as pl
from jax.experimental.pallas import tpu as pltpu
from jax.experimental.pallas import tpu_sc as plsc

_NC = 2
_NS = 16
_L = 16


def kernel(x, pe):
    batch, seq_len, dim = x.shape
    rows = batch * seq_len
    nw = _NC * _NS
    seq_w = seq_len // nw
    sch = 4
    n_ch = seq_w // sch
    chunk_el = sch * dim
    n_iter = (n_ch * batch) // 8
    nxb = 8

    mesh = plsc.VectorSubcoreMesh(core_axis_name="c", subcore_axis_name="s")

    scratch = (
        [pltpu.VMEM((chunk_el,), jnp.float32) for _ in range(nxb + 2)]
        + [pltpu.SemaphoreType.DMA for _ in range(2 * nxb + 2)]
    )

    @functools.partial(
        pl.kernel,
        mesh=mesh,
        out_type=jax.ShapeDtypeStruct((rows * dim,), jnp.float32),
        scratch_types=scratch,
    )
    def sc_add(x_hbm, pe_hbm, out_hbm, *scr):
        xb = scr[:nxb]
        peb = scr[nxb:nxb + 2]
        semx = scr[nxb + 2:2 * nxb + 2]
        semp = scr[2 * nxb + 2:2 * nxb + 4]
        semo = scr[2 * nxb + 4:3 * nxb + 4]

        wid = lax.axis_index("s") * _NC + lax.axis_index("c")
        s0 = wid * seq_w

        def x_off(g, local):
            ci = 2 * g + local // 4
            b = local % 4
            return (b * seq_len + s0 + ci * sch) * dim

        def pe_off(g, h):
            return (s0 + (2 * g + h) * sch) * dim

        def start_x(g, local, slot):
            pltpu.make_async_copy(
                x_hbm.at[pl.ds(x_off(g, local), chunk_el)], xb[slot], semx[slot]
            ).start()

        def wait_x(slot):
            pltpu.make_async_copy(
                x_hbm.at[pl.ds(0, chunk_el)], xb[slot], semx[slot]
            ).wait()

        def start_pe(g, h):
            pltpu.make_async_copy(
                pe_hbm.at[pl.ds(pe_off(g, h), chunk_el)], peb[h], semp[h]
            ).start()

        def wait_pe(h):
            pltpu.make_async_copy(
                pe_hbm.at[pl.ds(0, chunk_el)], peb[h], semp[h]
            ).wait()

        def start_out(g, local, slot):
            pltpu.make_async_copy(
                xb[slot], out_hbm.at[pl.ds(x_off(g, local), chunk_el)], semo[slot]
            ).start()

        def wait_out(slot):
            pltpu.make_async_copy(
                xb[slot], out_hbm.at[pl.ds(0, chunk_el)], semo[slot]
            ).wait()

        start_pe(0, 0)
        start_pe(0, 1)
        for sl in range(5):
            start_x(0, sl, sl)

        def iter_body(g, carry):
            for local in range(8):
                slot = local
                if local <= 2:
                    rs = local + 5

                    @pl.when(g > 0)
                    def _drain():
                        wait_out(rs)

                    start_x(g, rs, rs)
                else:
                    rs = local - 3

                    @pl.when(g + 1 < n_iter)
                    def _drain_fill():
                        wait_out(rs)
                        start_x(g + 1, rs, rs)

                if local == 0:
                    wait_pe(0)
                if local == 4:
                    wait_pe(1)
                wait_x(slot)

                xbuf = xb[slot]
                pb = peb[local // 4]

                @plsc.parallel_loop(0, chunk_el, step=8 * _L, unroll=4)
                def _vadd(o):
                    for u in range(8):
                        sl8 = pl.ds(o + u * _L, _L)
                        xbuf[sl8] = xbuf[sl8] + pb[sl8]

                start_out(g, local, slot)

                if local == 3:
                    @pl.when(g + 1 < n_iter)
                    def _pe0():
                        start_pe(g + 1, 0)
                if local == 7:
                    @pl.when(g + 1 < n_iter)
                    def _pe1():
                        start_pe(g + 1, 1)
            return carry

        lax.fori_loop(0, n_iter, iter_body, 0)
        for sl in range(nxb):
            wait_out(sl)

    out = sc_add(x.reshape(rows * dim), pe.reshape(seq_len * dim))
    return out.reshape(batch, seq_len, dim)


# speedup vs baseline: 1.3195x; 1.0024x over previous
import functools

import jax
import jax.numpy as jnp
from jax import lax
from jax.experimental import pallas as pl
from jax.experimental.pallas import tpu as pltpu
from jax.experimental.pallas import tpu_sc as plsc

_NC = 2
_NS = 16
_L = 16


def kernel(x, pe):
    batch, seq_len, dim = x.shape
    rows = batch * seq_len
    nw = _NC * _NS
    seq_w = seq_len // nw
    sch = 4
    n_ch = seq_w // sch
    chunk_el = sch * dim
    n_iter = (n_ch * batch) // 8
    nxb = 8

    mesh = plsc.VectorSubcoreMesh(core_axis_name="c", subcore_axis_name="s")

    scratch = (
        [pltpu.VMEM((chunk_el,), jnp.float32) for _ in range(nxb + 2)]
        + [pltpu.SemaphoreType.DMA for _ in range(2 * nxb + 2)]
    )

    @functools.partial(
        pl.kernel,
        mesh=mesh,
        out_type=jax.ShapeDtypeStruct((rows * dim,), jnp.float32),
        scratch_types=scratch,
    )
    def sc_add(x_hbm, pe_hbm, out_hbm, *scr):
        xb = scr[:nxb]
        peb = scr[nxb:nxb + 2]
        semx = scr[nxb + 2:2 * nxb + 2]
        semp = scr[2 * nxb + 2:2 * nxb + 4]
        semo = scr[2 * nxb + 4:3 * nxb + 4]

        wid = lax.axis_index("s") * _NC + lax.axis_index("c")
        s0 = wid * seq_w

        def x_off(g, local):
            ci = 2 * g + local // 4
            b = local % 4
            return (b * seq_len + s0 + ci * sch) * dim

        def pe_off(g, h):
            return (s0 + (2 * g + h) * sch) * dim

        def start_x(g, local, slot):
            pltpu.make_async_copy(
                x_hbm.at[pl.ds(x_off(g, local), chunk_el)], xb[slot], semx[slot]
            ).start()

        def wait_x(slot):
            pltpu.make_async_copy(
                x_hbm.at[pl.ds(0, chunk_el)], xb[slot], semx[slot]
            ).wait()

        def start_pe(g, h):
            pltpu.make_async_copy(
                pe_hbm.at[pl.ds(pe_off(g, h), chunk_el)], peb[h], semp[h]
            ).start()

        def wait_pe(h):
            pltpu.make_async_copy(
                pe_hbm.at[pl.ds(0, chunk_el)], peb[h], semp[h]
            ).wait()

        def start_out(g, local, slot):
            pltpu.make_async_copy(
                xb[slot], out_hbm.at[pl.ds(x_off(g, local), chunk_el)], semo[slot]
            ).start()

        def wait_out(slot):
            pltpu.make_async_copy(
                xb[slot], out_hbm.at[pl.ds(0, chunk_el)], semo[slot]
            ).wait()

        start_pe(0, 0)
        start_pe(0, 1)
        for sl in range(5):
            start_x(0, sl, sl)

        def iter_body(g, carry):
            for local in range(8):
                slot = local
                if local <= 2:
                    rs = local + 5

                    @pl.when(g > 0)
                    def _drain():
                        wait_out(rs)

                    start_x(g, rs, rs)
                else:
                    rs = local - 3

                    @pl.when(g + 1 < n_iter)
                    def _drain_fill():
                        wait_out(rs)
                        start_x(g + 1, rs, rs)

                if local == 0:
                    wait_pe(0)
                if local == 4:
                    wait_pe(1)
                wait_x(slot)

                xbuf = xb[slot]
                pb = peb[local // 4]

                @plsc.parallel_loop(0, chunk_el, step=4 * _L, unroll=2)
                def _vadd(o):
                    for u in range(4):
                        sl8 = pl.ds(o + u * _L, _L)
                        xbuf[sl8] = xbuf[sl8] + pb[sl8]

                start_out(g, local, slot)

                if local == 3:
                    @pl.when(g + 1 < n_iter)
                    def _pe0():
                        start_pe(g + 1, 0)
                if local == 7:
                    @pl.when(g + 1 < n_iter)
                    def _pe1():
                        start_pe(g + 1, 1)
            return carry

        lax.fori_loop(0, n_iter, iter_body, 0)
        for sl in range(nxb):
            wait_out(sl)

    out = sc_add(x.reshape(rows * dim), pe.reshape(seq_len * dim))
    return out.reshape(batch, seq_len, dim)


# speedup vs baseline: 5.4308x; 4.1158x over previous
import jax
import jax.numpy as jnp
from jax.experimental import pallas as pl


def _add_pe_kernel(x_ref, pe_ref, o_ref):
    o_ref[...] = x_ref[...] + pe_ref[...]


def kernel(x, pe):
    batch, seq_len, dim = x.shape
    sblk = 1024
    grid = (seq_len // sblk, batch)
    return pl.pallas_call(
        _add_pe_kernel,
        grid=grid,
        in_specs=[
            pl.BlockSpec((1, sblk, dim), lambda s, b: (b, s, 0)),
            pl.BlockSpec((sblk, dim), lambda s, b: (s, 0)),
        ],
        out_specs=pl.BlockSpec((1, sblk, dim), lambda s, b: (b, s, 0)),
        out_shape=jax.ShapeDtypeStruct(x.shape, x.dtype),
    )(x, pe)
